# baseline (speedup 1.0000x reference)
import jax
import jax.numpy as jnp
from jax import lax
from jax.experimental import pallas as pl
from jax.experimental.pallas import tpu as pltpu, tpu_sc as plsc

N = 10000
D = 128
E = 320000
ROUNDS = 3

NS = 16
CHUNK = 72
N_CH = 280
E_PAD = NS * N_CH * CHUNK
N_ACC = 10112
ZR = N_ACC // NS
NFC = N // CHUNK
TAIL = N - NFC * CHUNK
NBUF = 4


def _body(x, srcf, srcr, zacc, ones_h, zcnt,
          o0, o1, o2, o3, o4, o5,
          ip0, ip1, ip2, ip3, ip4, ip5, ip6, ip7,
          rows0, rows1, rows2, rows3, ones_v,
          accum_sh, cnt_sh, sem0, sem1, sem2, sem3,
          ssem0, ssem1, ssem2, ssem3, csem,
          isem0, isem1, isem2, isem3, isem4, isem5, isem6, isem7):
    cid = lax.axis_index("c")
    sid = lax.axis_index("s")
    ipair = [ip0, ip1, ip2, ip3, ip4, ip5, ip6, ip7]
    rows = [rows0, rows1, rows2, rows3]
    sems = [sem0, sem1, sem2, sem3]
    ssems = [ssem0, ssem1, ssem2, ssem3]
    isems = [isem0, isem1, isem2, isem3, isem4, isem5, isem6, isem7]
    NI = 8

    def scale_rows(buf, cbuf, nrows):
        def fin_body(rr, carry):
            cnt = cbuf[rr, :]
            inv = jnp.float32(1.0) / jnp.maximum(cnt, jnp.float32(1.0))
            for j in range(D // 16):
                buf[rr, pl.ds(j * 16, 16)] = buf[rr, pl.ds(j * 16, 16)] * inv
            return carry
        lax.fori_loop(0, nrows, fin_body, 0)

    def run(src_hbm, outs):
        pltpu.sync_copy(ones_h, ones_v)
        h = x
        for r in range(ROUNDS):
            pltpu.sync_copy(zacc, accum_sh.at[pl.ds(sid * ZR, ZR)])
            if r == 0:
                pltpu.sync_copy(zcnt, cnt_sh.at[pl.ds(sid * ZR, ZR)])
            plsc.subcore_barrier()

            for u in range(6):
                pltpu.make_async_copy(
                    src_hbm.at[sid, u], ipair[u], isems[u]).start()
            for u in range(2):
                pltpu.make_async_copy(
                    src_hbm.at[sid, u], ipair[u], isems[u]).wait()
                pltpu.make_async_copy(
                    h.at[ipair[u].at[0]], rows[u], sems[u]).start()

            def oct_body(i, carry):
                for u in range(2 * NBUF):
                    c = 2 * NBUF * i + u
                    q = u % NBUF
                    pq = (q + 2) % NBUF
                    i_c = u
                    i_m2 = (u + NI - 2) % NI
                    i_p2 = (u + 2) % NI
                    i_p6 = (u + 6) % NI

                    @pl.when(c >= 2)
                    def _():
                        pltpu.make_async_copy(
                            rows[pq], accum_sh.at[ipair[i_m2].at[1]],
                            ssems[pq]).wait()
                        if r == 0:
                            pltpu.make_async_copy(
                                ones_v, cnt_sh.at[ipair[i_m2].at[1]],
                                csem).wait()

                    @pl.when(c + 2 < N_CH)
                    def _():
                        pltpu.make_async_copy(
                            src_hbm.at[sid, c + 2], ipair[i_p2],
                            isems[i_p2]).wait()
                        pltpu.make_async_copy(
                            h.at[ipair[i_p2].at[0]], rows[pq],
                            sems[pq]).start()

                    @pl.when(c + 6 < N_CH)
                    def _():
                        pltpu.make_async_copy(
                            src_hbm.at[sid, c + 6], ipair[i_p6],
                            isems[i_p6]).start()

                    pltpu.make_async_copy(
                        h.at[ipair[i_c].at[0]], rows[q], sems[q]).wait()
                    pltpu.async_copy(
                        rows[q], accum_sh.at[ipair[i_c].at[1]],
                        ssems[q], add=True)
                    if r == 0:
                        pltpu.async_copy(
                            ones_v, cnt_sh.at[ipair[i_c].at[1]],
                            csem, add=True)
                return carry
            lax.fori_loop(0, N_CH // (2 * NBUF), oct_body, 0)
            for lc in (N_CH - 2, N_CH - 1):
                pltpu.make_async_copy(
                    rows[lc % NBUF], accum_sh.at[ipair[lc % NI].at[1]],
                    ssems[lc % NBUF]).wait()
                if r == 0:
                    pltpu.make_async_copy(
                        ones_v, cnt_sh.at[ipair[lc % NI].at[1]], csem).wait()
            plsc.subcore_barrier()

            for k in range(NFC // NS + 1):
                fc = sid + NS * k

                @pl.when(fc < NFC)
                def _():
                    c0 = fc * CHUNK
                    pltpu.sync_copy(accum_sh.at[pl.ds(c0, CHUNK)], rows0)
                    pltpu.sync_copy(cnt_sh.at[pl.ds(c0, CHUNK)], ones_v)
                    scale_rows(rows0, ones_v, CHUNK)
                    pltpu.sync_copy(rows0, outs[r].at[pl.ds(c0, CHUNK)])

            @pl.when(sid == NS - 1)
            def _():
                c0 = NFC * CHUNK
                pltpu.sync_copy(accum_sh.at[pl.ds(c0, TAIL)],
                                rows1.at[pl.ds(0, TAIL)])
                pltpu.sync_copy(cnt_sh.at[pl.ds(c0, TAIL)],
                                ones_v.at[pl.ds(0, TAIL)])
                scale_rows(rows1, ones_v, TAIL)
                pltpu.sync_copy(rows1.at[pl.ds(0, TAIL)],
                                outs[r].at[pl.ds(c0, TAIL)])

            plsc.subcore_barrier()
            h = outs[r]
            if r == 0:
                pltpu.sync_copy(ones_h, ones_v)

    @pl.when(cid == 0)
    def _():
        run(srcf, [o0, o1, o2])

    @pl.when(cid == 1)
    def _():
        run(srcr, [o3, o4, o5])


@jax.jit
def kernel(topic_entity_one_hot, edge_index, reverse_edge_index):
    x = topic_entity_one_hot

    def prep(ei):
        pad_src = jnp.zeros((E_PAD - E,), jnp.int32)
        pad_dst = jnp.full((E_PAD - E,), N, jnp.int32)
        src = jnp.concatenate([ei[0], pad_src]).reshape(NS, N_CH, 1, CHUNK)
        dst = jnp.concatenate([ei[1], pad_dst]).reshape(NS, N_CH, 1, CHUNK)
        return jnp.concatenate([src, dst], axis=2)

    srcf = prep(edge_index)
    srcr = prep(reverse_edge_index)
    zacc = jnp.zeros((ZR, D), jnp.float32)
    ones = jnp.ones((CHUNK, 16), jnp.float32)
    zcnt = jnp.zeros((ZR, 16), jnp.float32)

    out = jax.ShapeDtypeStruct((N, D), jnp.float32)
    mesh = plsc.VectorSubcoreMesh(core_axis_name="c", subcore_axis_name="s")
    fn = pl.kernel(
        _body,
        out_type=(out,) * 6,
        mesh=mesh,
        compiler_params=pltpu.CompilerParams(use_tc_tiling_on_sc=False),
        scratch_types=[
            pltpu.VMEM((2, CHUNK), jnp.int32),
            pltpu.VMEM((2, CHUNK), jnp.int32),
            pltpu.VMEM((2, CHUNK), jnp.int32),
            pltpu.VMEM((2, CHUNK), jnp.int32),
            pltpu.VMEM((2, CHUNK), jnp.int32),
            pltpu.VMEM((2, CHUNK), jnp.int32),
            pltpu.VMEM((2, CHUNK), jnp.int32),
            pltpu.VMEM((2, CHUNK), jnp.int32),
            pltpu.VMEM((CHUNK, D), jnp.float32),
            pltpu.VMEM((CHUNK, D), jnp.float32),
            pltpu.VMEM((CHUNK, D), jnp.float32),
            pltpu.VMEM((CHUNK, D), jnp.float32),
            pltpu.VMEM((CHUNK, 16), jnp.float32),
            pltpu.VMEM_SHARED((N_ACC, D), jnp.float32),
            pltpu.VMEM_SHARED((N_ACC, 16), jnp.float32),
            pltpu.SemaphoreType.DMA,
            pltpu.SemaphoreType.DMA,
            pltpu.SemaphoreType.DMA,
            pltpu.SemaphoreType.DMA,
            pltpu.SemaphoreType.DMA,
            pltpu.SemaphoreType.DMA,
            pltpu.SemaphoreType.DMA,
            pltpu.SemaphoreType.DMA,
            pltpu.SemaphoreType.DMA,
            pltpu.SemaphoreType.DMA,
            pltpu.SemaphoreType.DMA,
            pltpu.SemaphoreType.DMA,
            pltpu.SemaphoreType.DMA,
            pltpu.SemaphoreType.DMA,
            pltpu.SemaphoreType.DMA,
            pltpu.SemaphoreType.DMA,
            pltpu.SemaphoreType.DMA,
        ],
    )
    return fn(x, srcf, srcr, zacc, ones, zcnt)
